# baseline (speedup 1.0000x reference)
import functools

import jax
import jax.numpy as jnp
from jax import lax
from jax.experimental import pallas as pl
from jax.experimental.pallas import tpu as pltpu
from jax.experimental.pallas import tpu_sc as plsc

N = 10000
E = 320000
D = 128

NC = 2
NS = 16
NP = 10240
ROWS_PER_TILE = NP // NS
CHUNK = 64
NBUF = 4
NBUF_H = 4
EPT = 10240
EP = EPT * NC * NS
EDGES_PER_SC = EP // NC
NCHUNKS = EPT // CHUNK
NGROUPS = NCHUNKS // NBUF
CH = 128
NCHUNKS_H = EPT // CH
NGROUPS_H = NCHUNKS_H // NBUF_H

_mesh = plsc.VectorSubcoreMesh(core_axis_name="c", subcore_axis_name="s")

_IDX_RING = [pltpu.VMEM((CHUNK,), jnp.int32) for _ in range(NBUF)]
_IDX_RING_H = [pltpu.VMEM((CH,), jnp.int32) for _ in range(NBUF_H)]


@functools.partial(
    pl.kernel,
    out_type=jax.ShapeDtypeStruct((NC, NP), jnp.float32),
    mesh=_mesh,
    scratch_types=[
        pltpu.VMEM_SHARED((NP,), jnp.float32),
        *_IDX_RING_H,
        pltpu.VMEM((CH,), jnp.float32),
        pltpu.VMEM((CH,), jnp.float32),
        pltpu.SemaphoreType.DMA((NBUF_H,)),
        pltpu.SemaphoreType.DMA((NBUF_H,)),
    ],
)
def _deg_kernel(dst_hbm, out_hbm, dacc, d0, d1, d2, d3,
                ones_v, zeros_v, isem, ssem):
    cid = lax.axis_index("c")
    sid = lax.axis_index("s")
    dstv = [d0, d1, d2, d3]

    one = jnp.ones((16,), jnp.float32)
    zero = jnp.zeros((16,), jnp.float32)
    for j in range(CH // 16):
        ones_v[pl.ds(j * 16, 16)] = one
        zeros_v[pl.ds(j * 16, 16)] = zero

    for k in range(ROWS_PER_TILE // CH):
        pltpu.sync_copy(zeros_v, dacc.at[pl.ds(sid * ROWS_PER_TILE + k * CH, CH)])
    plsc.subcore_barrier()

    tile_base = cid * EDGES_PER_SC + sid * EPT

    def idx_load(c, b):
        pltpu.async_copy(dst_hbm.at[pl.ds(tile_base + c * CH, CH)],
                         dstv[b], isem.at[b])

    def idx_wait(b):
        pltpu.make_async_copy(dst_hbm.at[pl.ds(tile_base, CH)],
                              dstv[b], isem.at[b]).wait()

    def scatter(b):
        pltpu.async_copy(ones_v, dacc.at[dstv[b]], ssem.at[b], add=True)

    def scatter_wait(b):
        pltpu.make_async_copy(ones_v, dacc.at[dstv[b]], ssem.at[b]).wait()

    for b in range(NBUF_H):
        idx_load(b, b)

    def body(g, carry):
        for b in range(NBUF_H):
            idx_wait(b)
            scatter(b)
        for b in range(NBUF_H):
            scatter_wait(b)
            idx_load((g + 1) * NBUF_H + b, b)
        return carry

    lax.fori_loop(0, NGROUPS_H - 1, body, 0)
    for b in range(NBUF_H):
        idx_wait(b)
        scatter(b)
    for b in range(NBUF_H):
        scatter_wait(b)
    plsc.subcore_barrier()

    row0 = sid * ROWS_PER_TILE
    pltpu.sync_copy(dacc.at[pl.ds(row0, ROWS_PER_TILE)],
                    out_hbm.at[cid, pl.ds(row0, ROWS_PER_TILE)])


@functools.partial(
    pl.kernel,
    out_type=jax.ShapeDtypeStruct((NC, NP, D), jnp.float32),
    mesh=_mesh,
    scratch_types=[
        pltpu.VMEM_SHARED((NP, D), jnp.float32),
        pltpu.VMEM((EPT,), jnp.int32),
        *_IDX_RING,
        pltpu.VMEM((NBUF, CHUNK, D), jnp.float32),
        pltpu.SemaphoreType.DMA((NBUF,)),
        pltpu.SemaphoreType.DMA((NBUF,)),
        pltpu.SemaphoreType.DMA((NBUF,)),
    ],
)
def _agg_kernel(hs_hbm, src_hbm, dst_hbm, out_hbm,
                acc, src_all, d0, d1, d2, d3, rows, isem, gsem, ssem):
    cid = lax.axis_index("c")
    sid = lax.axis_index("s")
    dstv = [d0, d1, d2, d3]

    z = jnp.zeros((16,), jnp.float32)
    for r in range(16):
        for j in range(D // 16):
            rows[0, r, pl.ds(j * 16, 16)] = z

    tile_base = cid * EDGES_PER_SC + sid * EPT
    pltpu.async_copy(src_hbm.at[pl.ds(tile_base, EPT)], src_all, gsem.at[0])

    def zc(k, carry):
        pltpu.async_copy(rows.at[0].at[pl.ds(0, 16)],
                         acc.at[pl.ds(sid * ROWS_PER_TILE + k * 16, 16)],
                         ssem.at[0])
        return carry

    lax.fori_loop(0, ROWS_PER_TILE // 16, zc, 0)

    def zw(k, carry):
        pltpu.make_async_copy(rows.at[0].at[pl.ds(0, 16)],
                              acc.at[pl.ds(sid * ROWS_PER_TILE, 16)],
                              ssem.at[0]).wait()
        return carry

    lax.fori_loop(0, ROWS_PER_TILE // 16, zw, 0)
    pltpu.make_async_copy(src_hbm.at[pl.ds(tile_base, EPT)], src_all,
                          gsem.at[0]).wait()
    plsc.subcore_barrier()

    def idx_load(c, b):
        pltpu.async_copy(dst_hbm.at[pl.ds(tile_base + c * CHUNK, CHUNK)],
                         dstv[b], isem.at[b])

    def idx_wait(b):
        pltpu.make_async_copy(dst_hbm.at[pl.ds(tile_base, CHUNK)],
                              dstv[b], isem.at[b]).wait()

    def gather(c, b):
        pltpu.async_copy(hs_hbm.at[src_all.at[pl.ds(c * CHUNK, CHUNK)]],
                         rows.at[b], gsem.at[b])

    def gather_wait(b):
        pltpu.make_async_copy(hs_hbm.at[src_all.at[pl.ds(0, CHUNK)]],
                              rows.at[b], gsem.at[b]).wait()

    def scatter(b):
        pltpu.async_copy(rows.at[b], acc.at[dstv[b]], ssem.at[b], add=True)

    def scatter_wait(b):
        pltpu.make_async_copy(rows.at[b], acc.at[dstv[b]], ssem.at[b]).wait()

    for b in range(NBUF):
        idx_load(b, b)
        gather(b, b)

    def body(g, carry):
        for b in range(NBUF):
            gather_wait(b)
            idx_wait(b)
            scatter(b)
        for b in range(NBUF):
            cn = (g + 1) * NBUF + b
            scatter_wait(b)
            gather(cn, b)
            idx_load(cn, b)
        return carry

    lax.fori_loop(0, NGROUPS - 1, body, 0)

    for b in range(NBUF):
        gather_wait(b)
        idx_wait(b)
        scatter(b)
    for b in range(NBUF):
        scatter_wait(b)
    plsc.subcore_barrier()

    row0 = sid * ROWS_PER_TILE
    pltpu.sync_copy(acc.at[pl.ds(row0, ROWS_PER_TILE)],
                    out_hbm.at[cid, pl.ds(row0, ROWS_PER_TILE)])


_BN = 10000
_BO = 400


def _matmul_body(x_ref, w_ref, dv_ref, b_ref, o_ref, xb_ref):
    hs = jnp.dot(x_ref[...], w_ref[...],
                 preferred_element_type=jnp.float32) * dv_ref[...]
    o_ref[...] = hs
    xb_ref[...] = x_ref[...] + b_ref[...] + dv_ref[...] * hs


def _out_body(xb_ref, dv_ref, s0_ref, s1_ref, o_ref):
    o_ref[...] = xb_ref[...] + dv_ref[...] * (s0_ref[0] + s1_ref[0])


def kernel(x, edge_index, W, b):
    pad = EP - E
    src = jnp.concatenate(
        [edge_index[0], jnp.arange(pad, dtype=jnp.int32) % N])
    dst = jnp.concatenate(
        [edge_index[1], N + (jnp.arange(pad, dtype=jnp.int32) % (NP - N))])

    row_spec_m = pl.BlockSpec((_BN, D), lambda i: (i, 0))
    dp = _deg_kernel(dst)
    deg = 1.0 + dp[0, :N] + dp[1, :N]
    dinvb = jnp.broadcast_to(lax.rsqrt(deg)[:, None], (N, D))
    hs, xb = pl.pallas_call(
        _matmul_body,
        grid=(N // _BN,),
        in_specs=[row_spec_m,
                  pl.BlockSpec((D, D), lambda i: (0, 0)),
                  row_spec_m,
                  pl.BlockSpec((1, D), lambda i: (0, 0))],
        out_specs=[row_spec_m, row_spec_m],
        out_shape=[jax.ShapeDtypeStruct((N, D), jnp.float32),
                   jax.ShapeDtypeStruct((N, D), jnp.float32)],
    )(x, W, dinvb, b.reshape(1, D))

    sp = _agg_kernel(hs, src, dst)

    row_spec = pl.BlockSpec((_BO, D), lambda i: (i, 0))
    sp_spec0 = pl.BlockSpec((1, _BO, D), lambda i: (0, i, 0))
    sp_spec1 = pl.BlockSpec((1, _BO, D), lambda i: (1, i, 0))
    out = pl.pallas_call(
        _out_body,
        grid=(N // _BO,),
        in_specs=[row_spec, row_spec, sp_spec0, sp_spec1],
        out_specs=row_spec,
        out_shape=jax.ShapeDtypeStruct((N, D), jnp.float32),
    )(xb, dinvb, sp, sp)

    return out


# speedup vs baseline: 1.0298x; 1.0298x over previous
import functools

import jax
import jax.numpy as jnp
from jax import lax
from jax.experimental import pallas as pl
from jax.experimental.pallas import tpu as pltpu
from jax.experimental.pallas import tpu_sc as plsc

N = 10000
E = 320000
D = 128

NC = 2
NS = 16
NP = 10240
ROWS_PER_TILE = NP // NS
CHUNK = 64
NBUF = 4
NBUF_H = 8
EPT = 10240
EP = EPT * NC * NS
EDGES_PER_SC = EP // NC
NCHUNKS = EPT // CHUNK
NGROUPS = NCHUNKS // NBUF
CH = 128
NCHUNKS_H = EPT // CH
NGROUPS_H = NCHUNKS_H // NBUF_H

_mesh = plsc.VectorSubcoreMesh(core_axis_name="c", subcore_axis_name="s")

_IDX_RING = [pltpu.VMEM((CHUNK,), jnp.int32) for _ in range(NBUF)]
_IDX_RING_H = [pltpu.VMEM((CH,), jnp.int32) for _ in range(NBUF_H)]


@functools.partial(
    pl.kernel,
    out_type=jax.ShapeDtypeStruct((NC, NP), jnp.float32),
    mesh=_mesh,
    scratch_types=[
        pltpu.VMEM_SHARED((NP,), jnp.float32),
        *_IDX_RING_H,
        pltpu.VMEM((CH,), jnp.float32),
        pltpu.VMEM((CH,), jnp.float32),
        pltpu.SemaphoreType.DMA((NBUF_H,)),
        pltpu.SemaphoreType.DMA((NBUF_H,)),
    ],
)
def _deg_kernel(dst_hbm, out_hbm, dacc, d0, d1, d2, d3, d4, d5, d6, d7,
                ones_v, zeros_v, isem, ssem):
    cid = lax.axis_index("c")
    sid = lax.axis_index("s")
    dstv = [d0, d1, d2, d3, d4, d5, d6, d7]

    one = jnp.ones((16,), jnp.float32)
    zero = jnp.zeros((16,), jnp.float32)
    for j in range(CH // 16):
        ones_v[pl.ds(j * 16, 16)] = one
        zeros_v[pl.ds(j * 16, 16)] = zero

    for k in range(ROWS_PER_TILE // CH):
        pltpu.sync_copy(zeros_v, dacc.at[pl.ds(sid * ROWS_PER_TILE + k * CH, CH)])
    plsc.subcore_barrier()

    tile_base = cid * EDGES_PER_SC + sid * EPT

    def idx_load(c, b):
        pltpu.async_copy(dst_hbm.at[pl.ds(tile_base + c * CH, CH)],
                         dstv[b], isem.at[b])

    def idx_wait(b):
        pltpu.make_async_copy(dst_hbm.at[pl.ds(tile_base, CH)],
                              dstv[b], isem.at[b]).wait()

    def scatter(b):
        pltpu.async_copy(ones_v, dacc.at[dstv[b]], ssem.at[b], add=True)

    def scatter_wait(b):
        pltpu.make_async_copy(ones_v, dacc.at[dstv[b]], ssem.at[b]).wait()

    for b in range(NBUF_H):
        idx_load(b, b)

    def body(g, carry):
        for b in range(NBUF_H):
            idx_wait(b)
            scatter(b)
        for b in range(NBUF_H):
            scatter_wait(b)
            idx_load((g + 1) * NBUF_H + b, b)
        return carry

    lax.fori_loop(0, NGROUPS_H - 1, body, 0)
    for b in range(NBUF_H):
        idx_wait(b)
        scatter(b)
    for b in range(NBUF_H):
        scatter_wait(b)
    plsc.subcore_barrier()

    row0 = sid * ROWS_PER_TILE
    pltpu.sync_copy(dacc.at[pl.ds(row0, ROWS_PER_TILE)],
                    out_hbm.at[cid, pl.ds(row0, ROWS_PER_TILE)])


@functools.partial(
    pl.kernel,
    out_type=jax.ShapeDtypeStruct((NC, NP, D), jnp.float32),
    mesh=_mesh,
    scratch_types=[
        pltpu.VMEM_SHARED((NP, D), jnp.float32),
        pltpu.VMEM((EPT,), jnp.int32),
        *_IDX_RING,
        pltpu.VMEM((NBUF, CHUNK, D), jnp.float32),
        pltpu.SemaphoreType.DMA((NBUF,)),
        pltpu.SemaphoreType.DMA((NBUF,)),
        pltpu.SemaphoreType.DMA((NBUF,)),
    ],
)
def _agg_kernel(hs_hbm, src_hbm, dst_hbm, out_hbm,
                acc, src_all, d0, d1, d2, d3, rows, isem, gsem, ssem):
    cid = lax.axis_index("c")
    sid = lax.axis_index("s")
    dstv = [d0, d1, d2, d3]

    z = jnp.zeros((16,), jnp.float32)
    for r in range(16):
        for j in range(D // 16):
            rows[0, r, pl.ds(j * 16, 16)] = z

    tile_base = cid * EDGES_PER_SC + sid * EPT
    pltpu.async_copy(src_hbm.at[pl.ds(tile_base, EPT)], src_all, gsem.at[0])

    def zc(k, carry):
        pltpu.async_copy(rows.at[0].at[pl.ds(0, 16)],
                         acc.at[pl.ds(sid * ROWS_PER_TILE + k * 16, 16)],
                         ssem.at[0])
        return carry

    lax.fori_loop(0, ROWS_PER_TILE // 16, zc, 0)

    def zw(k, carry):
        pltpu.make_async_copy(rows.at[0].at[pl.ds(0, 16)],
                              acc.at[pl.ds(sid * ROWS_PER_TILE, 16)],
                              ssem.at[0]).wait()
        return carry

    lax.fori_loop(0, ROWS_PER_TILE // 16, zw, 0)
    pltpu.make_async_copy(src_hbm.at[pl.ds(tile_base, EPT)], src_all,
                          gsem.at[0]).wait()
    plsc.subcore_barrier()

    def idx_load(c, b):
        pltpu.async_copy(dst_hbm.at[pl.ds(tile_base + c * CHUNK, CHUNK)],
                         dstv[b], isem.at[b])

    def idx_wait(b):
        pltpu.make_async_copy(dst_hbm.at[pl.ds(tile_base, CHUNK)],
                              dstv[b], isem.at[b]).wait()

    def gather(c, b):
        pltpu.async_copy(hs_hbm.at[src_all.at[pl.ds(c * CHUNK, CHUNK)]],
                         rows.at[b], gsem.at[b])

    def gather_wait(b):
        pltpu.make_async_copy(hs_hbm.at[src_all.at[pl.ds(0, CHUNK)]],
                              rows.at[b], gsem.at[b]).wait()

    def scatter(b):
        pltpu.async_copy(rows.at[b], acc.at[dstv[b]], ssem.at[b], add=True)

    def scatter_wait(b):
        pltpu.make_async_copy(rows.at[b], acc.at[dstv[b]], ssem.at[b]).wait()

    for b in range(NBUF):
        idx_load(b, b)
        gather(b, b)

    def body(g, carry):
        for b in range(NBUF):
            gather_wait(b)
            idx_wait(b)
            scatter(b)
        for b in range(NBUF):
            cn = (g + 1) * NBUF + b
            scatter_wait(b)
            gather(cn, b)
            idx_load(cn, b)
        return carry

    lax.fori_loop(0, NGROUPS - 1, body, 0)

    for b in range(NBUF):
        gather_wait(b)
        idx_wait(b)
        scatter(b)
    for b in range(NBUF):
        scatter_wait(b)
    plsc.subcore_barrier()

    row0 = sid * ROWS_PER_TILE
    pltpu.sync_copy(acc.at[pl.ds(row0, ROWS_PER_TILE)],
                    out_hbm.at[cid, pl.ds(row0, ROWS_PER_TILE)])


_BN = 10000
_BO = 400


def _matmul_body(x_ref, w_ref, dv_ref, b_ref, o_ref, xb_ref):
    hs = jnp.dot(x_ref[...], w_ref[...],
                 preferred_element_type=jnp.float32) * dv_ref[...]
    o_ref[...] = hs
    xb_ref[...] = x_ref[...] + b_ref[...] + dv_ref[...] * hs


def _out_body(xb_ref, dv_ref, s0_ref, s1_ref, o_ref):
    o_ref[...] = xb_ref[...] + dv_ref[...] * (s0_ref[0] + s1_ref[0])


def kernel(x, edge_index, W, b):
    pad = EP - E
    src = jnp.concatenate(
        [edge_index[0], jnp.arange(pad, dtype=jnp.int32) % N])
    dst = jnp.concatenate(
        [edge_index[1], N + (jnp.arange(pad, dtype=jnp.int32) % (NP - N))])

    row_spec_m = pl.BlockSpec((_BN, D), lambda i: (i, 0))
    dp = _deg_kernel(dst)
    deg = 1.0 + dp[0, :N] + dp[1, :N]
    dinvb = jnp.broadcast_to(lax.rsqrt(deg)[:, None], (N, D))
    hs, xb = pl.pallas_call(
        _matmul_body,
        grid=(N // _BN,),
        in_specs=[row_spec_m,
                  pl.BlockSpec((D, D), lambda i: (0, 0)),
                  row_spec_m,
                  pl.BlockSpec((1, D), lambda i: (0, 0))],
        out_specs=[row_spec_m, row_spec_m],
        out_shape=[jax.ShapeDtypeStruct((N, D), jnp.float32),
                   jax.ShapeDtypeStruct((N, D), jnp.float32)],
    )(x, W, dinvb, b.reshape(1, D))

    sp = _agg_kernel(hs, src, dst)

    row_spec = pl.BlockSpec((_BO, D), lambda i: (i, 0))
    sp_spec0 = pl.BlockSpec((1, _BO, D), lambda i: (0, i, 0))
    sp_spec1 = pl.BlockSpec((1, _BO, D), lambda i: (1, i, 0))
    out = pl.pallas_call(
        _out_body,
        grid=(N // _BO,),
        in_specs=[row_spec, row_spec, sp_spec0, sp_spec1],
        out_specs=row_spec,
        out_shape=jax.ShapeDtypeStruct((N, D), jnp.float32),
    )(xb, dinvb, sp, sp)

    return out
